# baseline (speedup 1.0000x reference)
import numpy as np
import jax
import jax.numpy as jnp
from jax import lax
from jax.experimental import pallas as pl
from jax.experimental.pallas import tpu as pltpu
from jax.experimental.pallas import tpu_sc as plsc

_BXC_XY = float(np.float32(np.float32(-51.2 + 0.2) - np.float32(0.4) / np.float32(2.0)))
_DX_XY = float(np.float32(0.4))
_BXC_Z = float(np.float32(np.float32(-5.0 + 4.0) - np.float32(8.0) / np.float32(2.0)))
_DX_Z = float(np.float32(8.0))

_NB = 4
_NPB = 120000
_NXY = 65536
_CB = 64
_NC, _NS, _L = 2, 16, 16
_TPB = 8
_NPT = _NPB // _TPB
_CH = 2048
_NFULL = _NPT // _CH
_TAIL = _NPT - _NFULL * _CH
_TAILV = (_TAIL + _L - 1) // _L
_VPT = _NXY // _TPB
_VCH = 2048


def _sc_body(ptab, gout, wout, ghbm, grid, xbuf, ybuf, zbuf, idxbuf, gbuf,
             wbuf, tmpbuf, sem):
    core = lax.axis_index("c")
    sub = lax.axis_index("s")
    batch = core * 2 + sub // 8
    gpos = sub % 8
    lane = lax.broadcasted_iota(jnp.int32, (_L,), 0)

    with jax.named_scope("sc_init"):
        neg1 = jnp.full((_L,), -1, jnp.int32)

        def init_step(i, _):
            for u in range(8):
                grid[pl.ds(i * (8 * _L) + u * _L, _L)] = neg1
            return 0
        lax.fori_loop(0, _NXY // (8 * _L), init_step, 0)

    tile_start = batch * _NPB + gpos * _NPT

    def point_vregs(nbase, nv, limit):
        def vstep(i, _):
            rowi = i * _L + lane
            s16 = pl.ds(i * _L, _L)
            x = xbuf[s16]
            y = ybuf[s16]
            z = zbuf[s16]
            vx = (x - _BXC_XY) / _DX_XY
            vy = (y - _BXC_XY) / _DX_XY
            vz = (z - _BXC_Z) / _DX_Z
            valid = ((vx > -1.0) & (vx < 256.0) & (vy > -1.0) & (vy < 256.0)
                     & (vz > -1.0) & (vz < 1.0))
            if limit is not None:
                valid = valid & (rowi < limit)
            cx = jnp.clip(vx, -1.0, 256.0).astype(jnp.int32)
            cy = jnp.clip(vy, -1.0, 256.0).astype(jnp.int32)
            ridx = (cy * 256 + cx) & (_NXY - 1)
            nglob = nbase + rowi
            plsc.store_scatter(grid, [ridx], nglob, mask=valid)
            return 0
        lax.fori_loop(0, nv, vstep, 0)

    with jax.named_scope("sc_phaseA"):
        bofs = batch * (_NPB * 4)
        nstart = gpos * _NPT

        def do_chunk(c, _):
            nb = nstart + c * _CH
            for k, buf in ((0, xbuf), (1, ybuf), (2, zbuf)):
                pltpu.sync_copy(ptab.at[pl.ds(bofs + k * _NPB + nb, _CH)], buf)
            point_vregs(nb, _CH // _L, None)
            return 0
        lax.fori_loop(0, _NFULL, do_chunk, 0)
        nb_t = nstart + _NFULL * _CH
        for k, buf in ((0, xbuf), (1, ybuf), (2, zbuf)):
            pltpu.sync_copy(ptab.at[pl.ds(bofs + k * _NPB + nb_t, _TAIL)],
                            buf.at[pl.ds(0, _TAIL)])
        point_vregs(nb_t, _TAILV, _TAIL)

    with jax.named_scope("sc_phaseB"):
        gid = core * _NS + sub
        pltpu.sync_copy(grid, ghbm.at[gid])
        plsc.subcore_barrier()
        rowbase = core * _NS + (sub // 8) * 8
        vs = gpos * _VPT
        pltpu.sync_copy(ghbm.at[rowbase, pl.ds(vs, _VPT)], wbuf)

        def merge_one(j, _):
            pltpu.sync_copy(ghbm.at[rowbase + j, pl.ds(vs, _VPT)], tmpbuf)

            def mstep(i, _):
                for u in range(8):
                    s = pl.ds(i * (8 * _L) + u * _L, _L)
                    wbuf[s] = jnp.maximum(wbuf[s], tmpbuf[s])
                return 0
            lax.fori_loop(0, _VPT // (8 * _L), mstep, 0)
            return 0
        lax.fori_loop(1, _TPB, merge_one, 0)
        pltpu.sync_copy(wbuf, wout.at[batch, 0, pl.ds(vs, _VPT)])

    with jax.named_scope("sc_phaseC"):
        _run_phase_c(ptab, gout, batch, vs, wbuf, idxbuf, gbuf, sem)


def _run_phase_c(ptab, gout, batch, vs, wbuf, idxbuf, gbuf, sem):
    lane = lax.broadcasted_iota(jnp.int32, (_L,), 0)

    def do_voxchunk(sc_i, _):
        vb = vs + sc_i * _VCH

        def kloop(k, _):
            kofs = batch * (_NPB * 4) + k * _NPB

            def istep(i, _):
                w16 = wbuf[pl.ds(sc_i * _VCH + i * _L, _L)]
                dummy = vb + i * _L + lane
                row = jnp.where(w16 >= 0, w16, dummy)
                idxv = kofs + row
                idxbuf[k * 16 + i // 8, pl.ds((i % 8) * _L, _L)] = idxv
                return 0
            lax.fori_loop(0, _VCH // _L, istep, 0)
            for j in range(16):
                pltpu.async_copy(ptab.at[idxbuf.at[k * 16 + j]],
                                 gbuf.at[k, pl.ds(j * 128, 128)], sem)
            return 0
        lax.fori_loop(0, 4, kloop, 0)

        def drain(d, _):
            pltpu.make_async_copy(
                ptab.at[pl.ds(0, 128)], gbuf.at[0, pl.ds(0, 128)], sem).wait()
            return 0
        lax.fori_loop(0, 64, drain, 0)
        pltpu.sync_copy(gbuf, gout.at[batch, :, pl.ds(vb, _VCH)])
        return 0
    lax.fori_loop(0, _VPT // _VCH, do_voxchunk, 0)


_sc_stage = pl.kernel(
    _sc_body,
    out_type=(jax.ShapeDtypeStruct((_NB, 4, _NXY), jnp.float32),
              jax.ShapeDtypeStruct((_NB, 1, _NXY), jnp.int32),
              jax.ShapeDtypeStruct((_NC * _NS, _NXY), jnp.int32)),
    mesh=plsc.VectorSubcoreMesh(core_axis_name="c", subcore_axis_name="s",
                                num_cores=_NC, num_subcores=_NS),
    compiler_params=pltpu.CompilerParams(needs_layout_passes=False),
    scratch_types=[
        pltpu.VMEM((_NXY,), jnp.int32),
        pltpu.VMEM((_CH,), jnp.float32),
        pltpu.VMEM((_CH,), jnp.float32),
        pltpu.VMEM((_CH,), jnp.float32),
        pltpu.VMEM((64, 128), jnp.int32),
        pltpu.VMEM((4, _VCH), jnp.float32),
        pltpu.VMEM((_VPT,), jnp.int32),
        pltpu.VMEM((_VPT,), jnp.int32),
        pltpu.SemaphoreType.DMA,
    ],
)


def _tc_body(w2_ref, b2_ref, g_ref, m_ref, o_ref):
    gblk = g_ref[0]
    msk = m_ref[0] >= 0
    w2 = w2_ref[...]
    acc = b2_ref[...] + w2[:, 0:1] * gblk[0:1]
    acc = acc + w2[:, 1:2] * gblk[1:2]
    acc = acc + w2[:, 2:3] * gblk[2:3]
    acc = acc + w2[:, 3:4] * gblk[3:4]
    acc = jnp.maximum(acc, 0.0)
    res = jnp.where(msk, acc, 0.0)
    o_ref[0] = res.reshape(_CB, _TCV // 256, 256)


_TCV = 8192
_tc_stage = pl.pallas_call(
    _tc_body,
    grid=(_NB, _NXY // _TCV),
    in_specs=[
        pl.BlockSpec((_CB, 4), lambda bi, vi: (0, 0)),
        pl.BlockSpec((_CB, 1), lambda bi, vi: (0, 0)),
        pl.BlockSpec((1, 4, _TCV), lambda bi, vi: (bi, 0, vi)),
        pl.BlockSpec((1, 1, _TCV), lambda bi, vi: (bi, 0, vi)),
    ],
    out_specs=pl.BlockSpec((1, _CB, _TCV // 256, 256),
                           lambda bi, vi: (bi, 0, vi, 0)),
    out_shape=jax.ShapeDtypeStruct((_NB, _CB, 256, 256), jnp.float32),
)


def kernel(points, W, b, gamma, beta, running_mean, running_var):
    B, N, C = points.shape
    pts_flat = jnp.transpose(points, (0, 2, 1)).reshape(B * C * N)
    gathered, winner, _ = _sc_stage(pts_flat)
    s = gamma / jnp.sqrt(running_var + 1e-5)
    W2 = W * s[:, None]
    b2 = ((b - running_mean) * s + beta)[:, None]
    return _tc_stage(W2, b2, gathered, winner)


# speedup vs baseline: 1.0150x; 1.0150x over previous
import numpy as np
import jax
import jax.numpy as jnp
from jax import lax
from jax.experimental import pallas as pl
from jax.experimental.pallas import tpu as pltpu
from jax.experimental.pallas import tpu_sc as plsc

_BXC_XY = float(np.float32(np.float32(-51.2 + 0.2) - np.float32(0.4) / np.float32(2.0)))
_DX_XY = float(np.float32(0.4))
_BXC_Z = float(np.float32(np.float32(-5.0 + 4.0) - np.float32(8.0) / np.float32(2.0)))
_DX_Z = float(np.float32(8.0))

_NB = 4
_NPB = 120000
_NXY = 65536
_CB = 64
_NC, _NS, _L = 2, 16, 16
_TPB = 8
_NPT = _NPB // _TPB
_CH = 2048
_NFULL = _NPT // _CH
_TAIL = _NPT - _NFULL * _CH
_TAILV = (_TAIL + _L - 1) // _L
_VPT = _NXY // _TPB
_VCH = 2048


def _sc_body(ptab, gout, wout, ghbm, grid, xbuf, ybuf, zbuf, idxbuf, gbuf,
             wbuf, tmpbuf, sem):
    core = lax.axis_index("c")
    sub = lax.axis_index("s")
    batch = core * 2 + sub // 8
    gpos = sub % 8
    lane = lax.broadcasted_iota(jnp.int32, (_L,), 0)

    with jax.named_scope("sc_init"):
        neg1 = jnp.full((_L,), -1, jnp.int32)

        def init_step(i, _):
            for u in range(8):
                grid[pl.ds(i * (8 * _L) + u * _L, _L)] = neg1
            return 0
        lax.fori_loop(0, _NXY // (8 * _L), init_step, 0)

    tile_start = batch * _NPB + gpos * _NPT

    def point_vregs(nbase, nv, limit):
        def vstep(i, _):
            rowi = i * _L + lane
            s16 = pl.ds(i * _L, _L)
            x = xbuf[s16]
            y = ybuf[s16]
            z = zbuf[s16]
            vx = (x - _BXC_XY) / _DX_XY
            vy = (y - _BXC_XY) / _DX_XY
            vz = (z - _BXC_Z) / _DX_Z
            valid = ((vx > -1.0) & (vx < 256.0) & (vy > -1.0) & (vy < 256.0)
                     & (vz > -1.0) & (vz < 1.0))
            if limit is not None:
                valid = valid & (rowi < limit)
            cx = jnp.clip(vx, -1.0, 256.0).astype(jnp.int32)
            cy = jnp.clip(vy, -1.0, 256.0).astype(jnp.int32)
            ridx = (cy * 256 + cx) & (_NXY - 1)
            nglob = nbase + rowi
            plsc.store_scatter(grid, [ridx], nglob, mask=valid)
            return 0
        lax.fori_loop(0, nv, vstep, 0)

    with jax.named_scope("sc_phaseA"):
        bofs = batch * (_NPB * 4)
        nstart = gpos * _NPT

        def do_chunk(c, _):
            nb = nstart + c * _CH
            for k, buf in ((0, xbuf), (1, ybuf), (2, zbuf)):
                pltpu.sync_copy(ptab.at[pl.ds(bofs + k * _NPB + nb, _CH)], buf)
            point_vregs(nb, _CH // _L, None)
            return 0
        lax.fori_loop(0, _NFULL, do_chunk, 0)
        nb_t = nstart + _NFULL * _CH
        for k, buf in ((0, xbuf), (1, ybuf), (2, zbuf)):
            pltpu.sync_copy(ptab.at[pl.ds(bofs + k * _NPB + nb_t, _TAIL)],
                            buf.at[pl.ds(0, _TAIL)])
        point_vregs(nb_t, _TAILV, _TAIL)

    with jax.named_scope("sc_phaseB"):
        gid = core * _NS + sub
        pltpu.sync_copy(grid, ghbm.at[gid])
        plsc.subcore_barrier()
        rowbase = core * _NS + (sub // 8) * 8
        vs = gpos * _VPT
        pltpu.sync_copy(ghbm.at[rowbase, pl.ds(vs, _VPT)], wbuf)

        def merge_one(j, _):
            pltpu.sync_copy(ghbm.at[rowbase + j, pl.ds(vs, _VPT)], tmpbuf)

            def mstep(i, _):
                for u in range(8):
                    s = pl.ds(i * (8 * _L) + u * _L, _L)
                    wbuf[s] = jnp.maximum(wbuf[s], tmpbuf[s])
                return 0
            lax.fori_loop(0, _VPT // (8 * _L), mstep, 0)
            return 0
        lax.fori_loop(1, _TPB, merge_one, 0)
        pltpu.sync_copy(wbuf, wout.at[batch, 0, pl.ds(vs, _VPT)])

    with jax.named_scope("sc_phaseC"):
        _run_phase_c(ptab, gout, batch, vs, wbuf, idxbuf, gbuf, sem)


def _run_phase_c(ptab, gout, batch, vs, wbuf, idxbuf, gbuf, sem):
    lane = lax.broadcasted_iota(jnp.int32, (_L,), 0)

    def do_voxchunk(sc_i, _):
        vb = vs + sc_i * _VCH

        def kloop(k, _):
            kofs = batch * (_NPB * 4) + k * _NPB

            def istep(i, _):
                w16 = wbuf[pl.ds(sc_i * _VCH + i * _L, _L)]
                dummy = vb + i * _L + lane
                row = jnp.where(w16 >= 0, w16, dummy)
                idxv = kofs + row
                idxbuf[k * 16 + i // 8, pl.ds((i % 8) * _L, _L)] = idxv
                return 0
            lax.fori_loop(0, _VCH // _L, istep, 0)
            for j in range(16):
                pltpu.async_copy(ptab.at[idxbuf.at[k * 16 + j]],
                                 gbuf.at[k, pl.ds(j * 128, 128)], sem)
            return 0
        lax.fori_loop(0, 4, kloop, 0)

        def drain(d, _):
            pltpu.make_async_copy(
                ptab.at[pl.ds(0, 128)], gbuf.at[0, pl.ds(0, 128)], sem).wait()
            return 0
        lax.fori_loop(0, 64, drain, 0)
        pltpu.sync_copy(gbuf, gout.at[batch, :, pl.ds(vb, _VCH)])
        return 0
    lax.fori_loop(0, _VPT // _VCH, do_voxchunk, 0)


_sc_stage = pl.kernel(
    _sc_body,
    out_type=(jax.ShapeDtypeStruct((_NB, 4, _NXY), jnp.float32),
              jax.ShapeDtypeStruct((_NB, 1, _NXY), jnp.int32),
              jax.ShapeDtypeStruct((_NC * _NS, _NXY), jnp.int32)),
    mesh=plsc.VectorSubcoreMesh(core_axis_name="c", subcore_axis_name="s",
                                num_cores=_NC, num_subcores=_NS),
    compiler_params=pltpu.CompilerParams(needs_layout_passes=False),
    scratch_types=[
        pltpu.VMEM((_NXY,), jnp.int32),
        pltpu.VMEM((_CH,), jnp.float32),
        pltpu.VMEM((_CH,), jnp.float32),
        pltpu.VMEM((_CH,), jnp.float32),
        pltpu.VMEM((64, 128), jnp.int32),
        pltpu.VMEM((4, _VCH), jnp.float32),
        pltpu.VMEM((_VPT,), jnp.int32),
        pltpu.VMEM((_VPT,), jnp.int32),
        pltpu.SemaphoreType.DMA,
    ],
)


def _tc_body(w2_ref, b2_ref, g_ref, m_ref, o_ref):
    gblk = g_ref[0]
    msk = m_ref[0] >= 0
    w2 = w2_ref[...]
    acc = b2_ref[...] + w2[:, 0:1] * gblk[0:1]
    acc = acc + w2[:, 1:2] * gblk[1:2]
    acc = acc + w2[:, 2:3] * gblk[2:3]
    acc = acc + w2[:, 3:4] * gblk[3:4]
    acc = jnp.maximum(acc, 0.0)
    res = jnp.where(msk, acc, 0.0)
    o_ref[0] = res.reshape(_CB, _TCV // 256, 256)


_TCV = 4096
_tc_stage = pl.pallas_call(
    _tc_body,
    grid=(_NB, _NXY // _TCV),
    in_specs=[
        pl.BlockSpec((_CB, 4), lambda bi, vi: (0, 0)),
        pl.BlockSpec((_CB, 1), lambda bi, vi: (0, 0)),
        pl.BlockSpec((1, 4, _TCV), lambda bi, vi: (bi, 0, vi)),
        pl.BlockSpec((1, 1, _TCV), lambda bi, vi: (bi, 0, vi)),
    ],
    out_specs=pl.BlockSpec((1, _CB, _TCV // 256, 256),
                           lambda bi, vi: (bi, 0, vi, 0)),
    out_shape=jax.ShapeDtypeStruct((_NB, _CB, 256, 256), jnp.float32),
)


def kernel(points, W, b, gamma, beta, running_mean, running_var):
    B, N, C = points.shape
    pts_flat = jnp.transpose(points, (0, 2, 1)).reshape(B * C * N)
    gathered, winner, _ = _sc_stage(pts_flat)
    s = gamma / jnp.sqrt(running_var + 1e-5)
    W2 = W * s[:, None]
    b2 = ((b - running_mean) * s + beta)[:, None]
    return _tc_stage(W2, b2, gathered, winner)
